# baseline (speedup 1.0000x reference)
import functools

import jax
import jax.numpy as jnp
from jax import lax
from jax.experimental import pallas as pl
from jax.experimental.pallas import tpu as pltpu
from jax.experimental.pallas import tpu_sc as plsc


def _rsqrt_newton(sv):
    ih = lax.bitcast_convert_type(sv, jnp.int32)
    ih = jnp.int32(0x5F3759DF) - lax.shift_right_logical(ih, 1)
    y = lax.bitcast_convert_type(ih, jnp.float32)
    for _ in range(3):
        y = y * (1.5 - 0.5 * sv * y * y)
    return y


def kernel(task_ids, table):
    B, = task_ids.shape
    V, D = table.shape
    info = plsc.get_sparse_core_info()
    NC, NS, L = info.num_cores, info.num_subcores, info.num_lanes
    NW = NC * NS
    b_per_w = B // NW
    CHUNK = 128
    n_chunks = b_per_w // CHUNK
    n_half = D // L

    mesh = plsc.VectorSubcoreMesh(core_axis_name="c", subcore_axis_name="s")

    @functools.partial(
        pl.kernel,
        out_type=jax.ShapeDtypeStruct((B, D), jnp.float32),
        mesh=mesh,
        compiler_params=pltpu.CompilerParams(
            needs_layout_passes=False, use_tc_tiling_on_sc=False),
        scratch_types=[
            pltpu.VMEM((n_chunks, CHUNK), jnp.int32),
            pltpu.VMEM((2, CHUNK, D), jnp.float32),
            pltpu.SemaphoreType.DMA,
            pltpu.SemaphoreType.DMA,
        ],
    )
    def sc_kernel(idx_hbm, table_hbm, out_hbm, idx_v, buf_v, gsem, osem):
        wid = lax.axis_index("s") * NC + lax.axis_index("c")
        base = wid * b_per_w

        for c in range(n_chunks):
            pltpu.sync_copy(idx_hbm.at[pl.ds(base + c * CHUNK, CHUNK)],
                            idx_v.at[c])

        def start_gather(c):
            return pltpu.async_copy(
                table_hbm.at[idx_v.at[c]], buf_v.at[c % 2], gsem)

        def normalize_chunk(slot):
            cbuf = buf_v.at[slot]

            def body(i, _):
                halves = [cbuf[i, pl.ds(h * L, L)] for h in range(n_half)]
                sq = halves[0] * halves[0]
                for h in range(1, n_half):
                    sq = sq + halves[h] * halves[h]
                s = jnp.sum(sq)
                sv = lax.broadcast_in_dim(s, (L,), ())
                sv = jnp.maximum(sv, 1e-24)
                y = _rsqrt_newton(sv)
                for h in range(n_half):
                    cbuf[i, pl.ds(h * L, L)] = halves[h] * y
                return 0

            lax.fori_loop(0, CHUNK, body, 0, unroll=4)

        gathers = [start_gather(0)]
        out_copies = []
        for c in range(n_chunks):
            gathers[c].wait()
            if c >= 1:
                out_copies[c - 1].wait()
            if c + 1 < n_chunks:
                gathers.append(start_gather(c + 1))
            normalize_chunk(c % 2)
            out_copies.append(pltpu.async_copy(
                buf_v.at[c % 2],
                out_hbm.at[pl.ds(base + c * CHUNK, CHUNK)], osem))
        out_copies[-1].wait()

    return sc_kernel(task_ids, table)
